# baseline (speedup 1.0000x reference)
import functools

import jax
import jax.numpy as jnp
from jax import lax
from jax.experimental import pallas as pl
from jax.experimental.pallas import tpu as pltpu

_NH = 12
_VS = 128
_BB = 2
_LOG2E = 1.4426950408889634


def _fused_attn_kernel(x_ref, w_ref, b_ref, mask_ref, o_ref,
                       q0_sc, k0_sc, v0_sc, b0_sc,
                       q1_sc, k1_sc, v1_sc, b1_sc,
                       *, num_heads, head_dim, hidden, causal_bias):
    T = x_ref.shape[1]
    H = hidden
    D = head_dim

    w = w_ref[...]
    b = b_ref[...]
    lane = lax.broadcasted_iota(jnp.int32, (T, num_heads * _VS), 1)
    ones_pat = jnp.where((lane & (_VS - 1)) >= D,
                         jnp.float32(1.0), jnp.float32(0.0)).astype(jnp.bfloat16)
    rows = lax.broadcasted_iota(jnp.int32, (T, T), 0)
    cols = lax.broadcasted_iota(jnp.int32, (T, T), 1)
    causal = jnp.where(cols > rows, jnp.float32(causal_bias), jnp.float32(0.0))

    qkv_scs = ((q0_sc, k0_sc, v0_sc), (q1_sc, k1_sc, v1_sc))
    bias_scs = (b0_sc, b1_sc)
    xs = {}

    def x_of(bb):
        if bb not in xs:
            xs[bb] = x_ref[bb].astype(jnp.bfloat16)
        return xs[bb]

    def proj_piece(bb, t):
        sc = qkv_scs[bb][t]
        y = jnp.dot(x_of(bb), w[:, t * H:(t + 1) * H],
                    preferred_element_type=jnp.float32) + b[:, t * H:(t + 1) * H]
        if t < 2:
            sc[...] = y.astype(jnp.bfloat16)
        else:
            sc[...] = ones_pat
            for h in range(num_heads):
                sc[:, h * _VS:h * _VS + D] = y[:, h * D:(h + 1) * D].astype(jnp.bfloat16)

    def make_bias(bb):
        bias = causal + mask_ref[bb, 0]
        stab = jnp.max(bias, axis=-1, keepdims=True)
        bias_scs[bb][...] = bias - stab

    def attn_head(bb, h):
        q_sc, k_sc, v_sc = qkv_scs[bb]
        sl = slice(h * D, (h + 1) * D)
        s = lax.dot_general(q_sc[:, sl], k_sc[:, sl], (((1,), (1,)), ((), ())),
                            preferred_element_type=jnp.float32)
        p = jnp.exp2(s + bias_scs[bb][...]).astype(jnp.bfloat16)
        accl = jnp.dot(p, v_sc[:, h * _VS:(h + 1) * _VS],
                       preferred_element_type=jnp.float32)
        inv_l = pl.reciprocal(accl[:, D:2 * D], approx=True)
        o_ref[bb, :, sl] = (accl[:, :D] * inv_l).astype(o_ref.dtype)

    make_bias(0)
    make_bias(1)
    for t in range(3):
        proj_piece(0, t)
    for h in range(num_heads):
        attn_head(0, h)
        if h in (1, 4, 7):
            proj_piece(1, (h - 1) // 3)
    for h in range(num_heads):
        attn_head(1, h)


def kernel(hidden_states, attention_mask, wq_t, wk_t, wv_t, bq, bk, bv):
    B, T, H = hidden_states.shape
    D = H // _NH
    inv_scale = float(D) ** -0.5
    q_fold = inv_scale * _LOG2E

    w_qkv = jnp.concatenate([wq_t * q_fold, wk_t, wv_t],
                            axis=1).astype(jnp.bfloat16)
    b_qkv = jnp.concatenate([bq * q_fold, bk, bv], axis=1).astype(jnp.float32)
    amask_scaled = (attention_mask * (inv_scale * _LOG2E)).astype(jnp.float32)
    amask_scaled = amask_scaled.reshape(B, 1, T)

    kern = functools.partial(
        _fused_attn_kernel,
        num_heads=_NH, head_dim=D, hidden=H,
        causal_bias=-1000.0 * inv_scale * _LOG2E)

    bsc = pltpu.VMEM((T, H), jnp.bfloat16)
    vsc = pltpu.VMEM((T, _NH * _VS), jnp.bfloat16)
    fsc = pltpu.VMEM((T, T), jnp.float32)

    return pl.pallas_call(
        kern,
        out_shape=jax.ShapeDtypeStruct((B, T, H), jnp.float32),
        grid_spec=pltpu.PrefetchScalarGridSpec(
            num_scalar_prefetch=0,
            grid=(B // _BB,),
            in_specs=[
                pl.BlockSpec((_BB, T, H), lambda b: (b, 0, 0)),
                pl.BlockSpec((H, 3 * H), lambda b: (0, 0)),
                pl.BlockSpec((1, 3 * H), lambda b: (0, 0)),
                pl.BlockSpec((_BB, 1, T), lambda b: (b, 0, 0)),
            ],
            out_specs=pl.BlockSpec((_BB, T, H), lambda b: (b, 0, 0)),
            scratch_shapes=[bsc, bsc, vsc, fsc, bsc, bsc, vsc, fsc],
        ),
        compiler_params=pltpu.CompilerParams(
            dimension_semantics=("parallel",)),
    )(hidden_states, w_qkv, b_qkv, amask_scaled)


# speedup vs baseline: 1.0222x; 1.0222x over previous
import functools

import jax
import jax.numpy as jnp
from jax import lax
from jax.experimental import pallas as pl
from jax.experimental.pallas import tpu as pltpu

_NH = 12
_VS = 128
_BB = 2


def _fused_attn_kernel(x_ref, w_ref, b_ref, mask_ref, o_ref,
                       q_sc, k_sc, v_sc, bias_sc,
                       *, num_heads, head_dim, hidden, causal_bias):
    T = x_ref.shape[1]
    H = hidden
    D = head_dim

    w = w_ref[...]
    b = b_ref[...]
    lane = lax.broadcasted_iota(jnp.int32, (T, num_heads * _VS), 1)
    ones_pat = jnp.where((lane & (_VS - 1)) >= D,
                         jnp.float32(1.0), jnp.float32(0.0)).astype(jnp.bfloat16)
    rows = lax.broadcasted_iota(jnp.int32, (T, T), 0)
    cols = lax.broadcasted_iota(jnp.int32, (T, T), 1)
    causal = jnp.where(cols > rows, jnp.float32(causal_bias), jnp.float32(0.0))

    for bb in range(_BB):
        x = x_ref[bb].astype(jnp.bfloat16)
        q = jnp.dot(x, w[:, :H], preferred_element_type=jnp.float32) + b[:, :H]
        q_sc[bb] = q.astype(jnp.bfloat16)
        k = jnp.dot(x, w[:, H:2 * H], preferred_element_type=jnp.float32) + b[:, H:2 * H]
        k_sc[bb] = k.astype(jnp.bfloat16)
        v = jnp.dot(x, w[:, 2 * H:], preferred_element_type=jnp.float32) + b[:, 2 * H:]

        v_sc[bb] = ones_pat
        for h in range(num_heads):
            v_sc[bb, :, h * _VS:h * _VS + D] = v[:, h * D:(h + 1) * D].astype(jnp.bfloat16)

        bias = causal + mask_ref[bb, 0]
        stab = jnp.max(bias, axis=-1, keepdims=True)
        bias_sc[bb] = bias - stab

    for bb in range(_BB):
        bias2 = bias_sc[bb]
        for h in range(num_heads):
            sl = slice(h * D, (h + 1) * D)
            s = lax.dot_general(q_sc[bb, :, sl], k_sc[bb, :, sl],
                                (((1,), (1,)), ((), ())),
                                preferred_element_type=jnp.float32)
            p = jnp.exp2(s + bias2).astype(jnp.bfloat16)
            accl = jnp.dot(p, v_sc[bb, :, h * _VS:(h + 1) * _VS],
                           preferred_element_type=jnp.float32)
            inv_l = pl.reciprocal(accl[:, D:2 * D], approx=True)
            o_ref[bb, :, sl] = (accl[:, :D] * inv_l).astype(o_ref.dtype)


def kernel(hidden_states, attention_mask, wq_t, wk_t, wv_t, bq, bk, bv):
    B, T, H = hidden_states.shape
    D = H // _NH
    inv_scale = float(D) ** -0.5
    q_fold = inv_scale * 1.4426950408889634

    w_qkv = jnp.concatenate([wq_t * q_fold, wk_t, wv_t],
                            axis=1).astype(jnp.bfloat16)
    b_qkv = jnp.concatenate([bq * q_fold, bk, bv], axis=1).astype(jnp.float32)
    amask_scaled = (attention_mask * q_fold).astype(jnp.float32)
    amask_scaled = amask_scaled.reshape(B, 1, T)

    kern = functools.partial(
        _fused_attn_kernel,
        num_heads=_NH, head_dim=D, hidden=H,
        causal_bias=-1000.0 * q_fold)

    return pl.pallas_call(
        kern,
        out_shape=jax.ShapeDtypeStruct((B, T, H), jnp.float32),
        grid_spec=pltpu.PrefetchScalarGridSpec(
            num_scalar_prefetch=0,
            grid=(B // _BB,),
            in_specs=[
                pl.BlockSpec((_BB, T, H), lambda b: (b, 0, 0)),
                pl.BlockSpec((H, 3 * H), lambda b: (0, 0)),
                pl.BlockSpec((1, 3 * H), lambda b: (0, 0)),
                pl.BlockSpec((_BB, 1, T), lambda b: (b, 0, 0)),
            ],
            out_specs=pl.BlockSpec((_BB, T, H), lambda b: (b, 0, 0)),
            scratch_shapes=[
                pltpu.VMEM((_BB, T, H), jnp.bfloat16),
                pltpu.VMEM((_BB, T, H), jnp.bfloat16),
                pltpu.VMEM((_BB, T, _NH * _VS), jnp.bfloat16),
                pltpu.VMEM((_BB, T, T), jnp.float32),
            ],
        ),
        compiler_params=pltpu.CompilerParams(
            dimension_semantics=("parallel",)),
    )(hidden_states, w_qkv, b_qkv, amask_scaled)


# speedup vs baseline: 1.0332x; 1.0108x over previous
import functools

import jax
import jax.numpy as jnp
from jax import lax
from jax.experimental import pallas as pl
from jax.experimental.pallas import tpu as pltpu

_NH = 12
_VS = 128
_BB = 4


def _fused_attn_kernel(x_ref, w_ref, b_ref, mask_ref, o_ref,
                       q_sc, k_sc, v_sc, bias_sc,
                       *, num_heads, head_dim, hidden, causal_bias):
    T = x_ref.shape[1]
    H = hidden
    D = head_dim

    w = w_ref[...]
    b = b_ref[...]
    lane = lax.broadcasted_iota(jnp.int32, (T, num_heads * _VS), 1)
    ones_pat = jnp.where((lane & (_VS - 1)) >= D,
                         jnp.float32(1.0), jnp.float32(0.0)).astype(jnp.bfloat16)
    rows = lax.broadcasted_iota(jnp.int32, (T, T), 0)
    cols = lax.broadcasted_iota(jnp.int32, (T, T), 1)
    causal = jnp.where(cols > rows, jnp.float32(causal_bias), jnp.float32(0.0))

    for bb in range(_BB):
        x = x_ref[bb].astype(jnp.bfloat16)
        q = jnp.dot(x, w[:, :H], preferred_element_type=jnp.float32) + b[:, :H]
        q_sc[bb] = q.astype(jnp.bfloat16)
        k = jnp.dot(x, w[:, H:2 * H], preferred_element_type=jnp.float32) + b[:, H:2 * H]
        k_sc[bb] = k.astype(jnp.bfloat16)
        v = jnp.dot(x, w[:, 2 * H:], preferred_element_type=jnp.float32) + b[:, 2 * H:]

        v_sc[bb] = ones_pat
        for h in range(num_heads):
            v_sc[bb, :, h * _VS:h * _VS + D] = v[:, h * D:(h + 1) * D].astype(jnp.bfloat16)

        bias = causal + mask_ref[bb, 0]
        stab = jnp.max(bias, axis=-1, keepdims=True)
        bias_sc[bb] = bias - stab

    for bb in range(_BB):
        bias2 = bias_sc[bb]
        for h in range(num_heads):
            sl = slice(h * D, (h + 1) * D)
            s = lax.dot_general(q_sc[bb, :, sl], k_sc[bb, :, sl],
                                (((1,), (1,)), ((), ())),
                                preferred_element_type=jnp.float32)
            p = jnp.exp2(s + bias2).astype(jnp.bfloat16)
            accl = jnp.dot(p, v_sc[bb, :, h * _VS:(h + 1) * _VS],
                           preferred_element_type=jnp.float32)
            inv_l = pl.reciprocal(accl[:, D:2 * D], approx=True)
            o_ref[bb, :, sl] = (accl[:, :D] * inv_l).astype(o_ref.dtype)


def kernel(hidden_states, attention_mask, wq_t, wk_t, wv_t, bq, bk, bv):
    B, T, H = hidden_states.shape
    D = H // _NH
    inv_scale = float(D) ** -0.5
    q_fold = inv_scale * 1.4426950408889634

    w_qkv = jnp.concatenate([wq_t * q_fold, wk_t, wv_t],
                            axis=1).astype(jnp.bfloat16)
    b_qkv = jnp.concatenate([bq * q_fold, bk, bv], axis=1).astype(jnp.float32)
    amask_scaled = (attention_mask * q_fold).astype(jnp.float32)
    amask_scaled = amask_scaled.reshape(B, 1, T)

    kern = functools.partial(
        _fused_attn_kernel,
        num_heads=_NH, head_dim=D, hidden=H,
        causal_bias=-1000.0 * q_fold)

    return pl.pallas_call(
        kern,
        out_shape=jax.ShapeDtypeStruct((B, T, H), jnp.float32),
        grid_spec=pltpu.PrefetchScalarGridSpec(
            num_scalar_prefetch=0,
            grid=(B // _BB,),
            in_specs=[
                pl.BlockSpec((_BB, T, H), lambda b: (b, 0, 0)),
                pl.BlockSpec((H, 3 * H), lambda b: (0, 0)),
                pl.BlockSpec((1, 3 * H), lambda b: (0, 0)),
                pl.BlockSpec((_BB, 1, T), lambda b: (b, 0, 0)),
            ],
            out_specs=pl.BlockSpec((_BB, T, H), lambda b: (b, 0, 0)),
            scratch_shapes=[
                pltpu.VMEM((_BB, T, H), jnp.bfloat16),
                pltpu.VMEM((_BB, T, H), jnp.bfloat16),
                pltpu.VMEM((_BB, T, _NH * _VS), jnp.bfloat16),
                pltpu.VMEM((_BB, T, T), jnp.float32),
            ],
        ),
        compiler_params=pltpu.CompilerParams(
            dimension_semantics=("parallel",)),
    )(hidden_states, w_qkv, b_qkv, amask_scaled)
